# Initial kernel scaffold; baseline (speedup 1.0000x reference)
import jax
import jax.numpy as jnp
from jax.experimental import pallas as pl


def kernel(x, edge_index, W, b):
    raise NotImplementedError("write your pallas kernel here")


# speedup vs baseline: 5.4226x; 5.4226x over previous
import functools

import jax
import jax.numpy as jnp
from jax import lax
from jax.experimental import pallas as pl
from jax.experimental.pallas import tpu as pltpu
from jax.experimental.pallas import tpu_sc as plsc

N = 10000
E = 320000
D = 128

NC = 2
NS = 16
NW = NC * NS

E_PER_W = E // NW
CHUNK = 80
NCHUNK = E_PER_W // CHUNK

ACC_ROWS = 10240
ZROWS = 640


def _linear_body(x_ref, w_ref, b_ref, o_ref):
    o_ref[...] = lax.dot_general(
        x_ref[...], w_ref[...],
        dimension_numbers=(((1,), (1,)), ((), ())),
        preferred_element_type=jnp.float32,
    ) + b_ref[...]


def _linear(x, W, b2):
    grid = 10
    rows = N // grid
    return pl.pallas_call(
        _linear_body,
        grid=(grid,),
        in_specs=[
            pl.BlockSpec((rows, D), lambda i: (i, 0)),
            pl.BlockSpec((D, D), lambda i: (0, 0)),
            pl.BlockSpec((1, D), lambda i: (0, 0)),
        ],
        out_specs=pl.BlockSpec((rows, D), lambda i: (i, 0)),
        out_shape=jax.ShapeDtypeStruct((N, D), jnp.float32),
    )(x, W, b2)


def _sc_body(sup_hbm, src_hbm, dst_hbm, out_hbm, acc, sidx_v, didx_v, rows_v,
             stage_v, zbuf_v, sem):
    cid = lax.axis_index("c")
    sid = lax.axis_index("s")
    wid = cid * NS + sid
    tid = sid

    for r in range(16):
        for j in range(8):
            zbuf_v[r, pl.ds(j * 16, 16)] = jnp.zeros((16,), jnp.float32)

    def _zero(i, carry):
        pltpu.sync_copy(zbuf_v, acc.at[pl.ds(tid * ZROWS + i * 16, 16)])
        return carry

    lax.fori_loop(0, ZROWS // 16, _zero, 0)
    plsc.subcore_barrier()

    base = wid * E_PER_W

    def _edge(c, carry):
        off = base + c * CHUNK
        pltpu.sync_copy(src_hbm.at[pl.ds(off, CHUNK)], sidx_v)
        pltpu.sync_copy(dst_hbm.at[pl.ds(off, CHUNK)], didx_v)
        pltpu.async_copy(sup_hbm.at[sidx_v], rows_v, sem).wait()
        pltpu.sync_copy(rows_v, acc.at[didx_v], add=True)
        return carry

    lax.fori_loop(0, NCHUNK, _edge, 0)
    plsc.subcore_barrier()

    def _wb(i, carry):
        r0 = tid * ZROWS + i * 128
        pltpu.sync_copy(acc.at[pl.ds(r0, 128)], stage_v)
        pltpu.sync_copy(stage_v, out_hbm.at[cid, pl.ds(r0, 128)])
        return carry

    lax.fori_loop(0, ZROWS // 128, _wb, 0)


@functools.cache
def _sc_aggregate():
    return pl.kernel(
        _sc_body,
        out_type=jax.ShapeDtypeStruct((NC, ACC_ROWS, D), jnp.float32),
        mesh=plsc.VectorSubcoreMesh(
            core_axis_name="c", subcore_axis_name="s",
            num_cores=NC, num_subcores=NS,
        ),
        scratch_types=[
            pltpu.VMEM_SHARED((ACC_ROWS, D), jnp.float32),
            pltpu.VMEM((CHUNK,), jnp.int32),
            pltpu.VMEM((CHUNK,), jnp.int32),
            pltpu.VMEM((CHUNK, D), jnp.float32),
            pltpu.VMEM((128, D), jnp.float32),
            pltpu.VMEM((16, D), jnp.float32),
            pltpu.SemaphoreType.DMA,
        ],
    )


def _combine_body(p_ref, o_ref):
    s = p_ref[0] + p_ref[1]
    o_ref[...] = jnp.where(s >= 0, s, 0.2 * s)


def _combine(partials):
    grid = 10
    rows = N // grid
    return pl.pallas_call(
        _combine_body,
        grid=(grid,),
        in_specs=[pl.BlockSpec((NC, rows, D), lambda i: (0, i, 0))],
        out_specs=pl.BlockSpec((rows, D), lambda i: (i, 0)),
        out_shape=jax.ShapeDtypeStruct((N, D), jnp.float32),
    )(partials)


def kernel(x, edge_index, W, b):
    support = _linear(x, W, b.reshape(1, D))
    partials = _sc_aggregate()(support, edge_index[0], edge_index[1])
    return _combine(partials)


# speedup vs baseline: 11.5593x; 2.1317x over previous
import functools

import jax
import jax.numpy as jnp
from jax import lax
from jax.experimental import pallas as pl
from jax.experimental.pallas import tpu as pltpu
from jax.experimental.pallas import tpu_sc as plsc

N = 10000
E = 320000
D = 128

NC = 2
NS = 16
NW = NC * NS

E_PER_W = E // NW
CHUNK = 80
NCHUNK = E_PER_W // CHUNK

ACC_ROWS = 10240
ZROWS = 640


def _linear_body(x_ref, w_ref, b_ref, o_ref):
    o_ref[...] = lax.dot_general(
        x_ref[...], w_ref[...],
        dimension_numbers=(((1,), (1,)), ((), ())),
        preferred_element_type=jnp.float32,
    ) + b_ref[...]


def _linear(x, W, b2):
    grid = 10
    rows = N // grid
    return pl.pallas_call(
        _linear_body,
        grid=(grid,),
        in_specs=[
            pl.BlockSpec((rows, D), lambda i: (i, 0)),
            pl.BlockSpec((D, D), lambda i: (0, 0)),
            pl.BlockSpec((1, D), lambda i: (0, 0)),
        ],
        out_specs=pl.BlockSpec((rows, D), lambda i: (i, 0)),
        out_shape=jax.ShapeDtypeStruct((N, D), jnp.float32),
    )(x, W, b2)


def _sc_body(sup_hbm, src_hbm, dst_hbm, out_hbm, acc, sidx_v, didx_v, rows0_v,
             rows1_v, isem, gsem0, gsem1):
    cid = lax.axis_index("c")
    sid = lax.axis_index("s")
    wid = cid * NS + sid
    tid = sid

    pltpu.async_copy(src_hbm.at[pl.ds(wid * E_PER_W, E_PER_W)], sidx_v, isem)
    pltpu.async_copy(dst_hbm.at[wid], didx_v, isem)

    def _zrow(r, carry):
        for j in range(8):
            rows0_v[r, pl.ds(j * 16, 16)] = jnp.zeros((16,), jnp.float32)
        return carry

    lax.fori_loop(0, CHUNK, _zrow, 0)

    def _zero(i, carry):
        pltpu.sync_copy(rows0_v, acc.at[pl.ds(tid * ZROWS + i * CHUNK, CHUNK)])
        return carry

    lax.fori_loop(0, ZROWS // CHUNK, _zero, 0)
    pltpu.make_async_copy(
        src_hbm.at[pl.ds(wid * E_PER_W, E_PER_W)], sidx_v, isem).wait()
    pltpu.make_async_copy(dst_hbm.at[wid], didx_v, isem).wait()
    plsc.subcore_barrier()

    def _gather(c, buf, sem):
        idx = sidx_v.at[pl.ds(c * CHUNK, CHUNK)]
        return pltpu.make_async_copy(sup_hbm.at[idx], buf, sem)

    def _scatter(c, buf):
        pltpu.sync_copy(buf, acc.at[didx_v.at[c]], add=True)

    _gather(0, rows0_v, gsem0).start()

    def _edge(i, carry):
        c0 = 2 * i
        _gather(c0 + 1, rows1_v, gsem1).start()
        _gather(c0, rows0_v, gsem0).wait()
        _scatter(c0, rows0_v)
        _gather(c0 + 2, rows0_v, gsem0).start()
        _gather(c0 + 1, rows1_v, gsem1).wait()
        _scatter(c0 + 1, rows1_v)
        return carry

    lax.fori_loop(0, (NCHUNK - 1) // 2, _edge, 0)
    _gather(NCHUNK - 1, rows0_v, gsem0).wait()
    _scatter(NCHUNK - 1, rows0_v)
    plsc.subcore_barrier()

    def _wb(i, carry):
        r0 = tid * ZROWS + i * CHUNK
        pltpu.sync_copy(acc.at[pl.ds(r0, CHUNK)], rows0_v)
        pltpu.sync_copy(rows0_v, out_hbm.at[cid, pl.ds(r0, CHUNK)])
        return carry

    lax.fori_loop(0, ZROWS // CHUNK, _wb, 0)


@functools.cache
def _sc_aggregate():
    return pl.kernel(
        _sc_body,
        out_type=jax.ShapeDtypeStruct((NC, ACC_ROWS, D), jnp.float32),
        mesh=plsc.VectorSubcoreMesh(
            core_axis_name="c", subcore_axis_name="s",
            num_cores=NC, num_subcores=NS,
        ),
        scratch_types=[
            pltpu.VMEM_SHARED((ACC_ROWS, D), jnp.float32),
            pltpu.VMEM((E_PER_W,), jnp.int32),
            pltpu.VMEM((NCHUNK, CHUNK), jnp.int32),
            pltpu.VMEM((CHUNK, D), jnp.float32),
            pltpu.VMEM((CHUNK, D), jnp.float32),
            pltpu.SemaphoreType.DMA,
            pltpu.SemaphoreType.DMA,
            pltpu.SemaphoreType.DMA,
        ],
    )


def _combine_body(p_ref, o_ref):
    s = p_ref[0] + p_ref[1]
    o_ref[...] = jnp.where(s >= 0, s, 0.2 * s)


def _combine(partials):
    grid = 10
    rows = N // grid
    return pl.pallas_call(
        _combine_body,
        grid=(grid,),
        in_specs=[pl.BlockSpec((NC, rows, D), lambda i: (0, i, 0))],
        out_specs=pl.BlockSpec((rows, D), lambda i: (i, 0)),
        out_shape=jax.ShapeDtypeStruct((N, D), jnp.float32),
    )(partials)


def kernel(x, edge_index, W, b):
    support = _linear(x, W, b.reshape(1, D))
    dst3 = edge_index[1].reshape(NW, NCHUNK, CHUNK)
    partials = _sc_aggregate()(support, edge_index[0], dst3)
    return _combine(partials)


# speedup vs baseline: 11.5971x; 1.0033x over previous
import functools

import jax
import jax.numpy as jnp
from jax import lax
from jax.experimental import pallas as pl
from jax.experimental.pallas import tpu as pltpu
from jax.experimental.pallas import tpu_sc as plsc

N = 10000
E = 320000
D = 128

NC = 2
NS = 16
NW = NC * NS

E_PER_W = E // NW
CHUNK = 80
NCHUNK = E_PER_W // CHUNK

ACC_ROWS = 10240
ZROWS = 640


def _linear_body(x_ref, w_ref, b_ref, o_ref):
    o_ref[...] = lax.dot_general(
        x_ref[...], w_ref[...],
        dimension_numbers=(((1,), (1,)), ((), ())),
        preferred_element_type=jnp.float32,
    ) + b_ref[...]


def _linear(x, W, b2):
    grid = 10
    rows = N // grid
    return pl.pallas_call(
        _linear_body,
        grid=(grid,),
        in_specs=[
            pl.BlockSpec((rows, D), lambda i: (i, 0)),
            pl.BlockSpec((D, D), lambda i: (0, 0)),
            pl.BlockSpec((1, D), lambda i: (0, 0)),
        ],
        out_specs=pl.BlockSpec((rows, D), lambda i: (i, 0)),
        out_shape=jax.ShapeDtypeStruct((N, D), jnp.float32),
    )(x, W, b2)


def _sc_body(sup_hbm, src_hbm, dst_hbm, out_hbm, acc, sidx_v, didx_v, rows0_v,
             rows1_v, isem, gsem0, gsem1, zsem):
    cid = lax.axis_index("c")
    sid = lax.axis_index("s")
    wid = cid * NS + sid
    tid = sid

    pltpu.async_copy(src_hbm.at[pl.ds(wid * E_PER_W, E_PER_W)], sidx_v, isem)
    pltpu.async_copy(dst_hbm.at[wid], didx_v, isem)

    def _zrow(r, carry):
        for j in range(8):
            rows0_v[r, pl.ds(j * 16, 16)] = jnp.zeros((16,), jnp.float32)
        return carry

    lax.fori_loop(0, CHUNK, _zrow, 0)

    def _zero(i, carry):
        pltpu.async_copy(
            rows0_v, acc.at[pl.ds(tid * ZROWS + i * CHUNK, CHUNK)], zsem)
        return carry

    lax.fori_loop(0, ZROWS // CHUNK, _zero, 0)

    def _zdrain(i, carry):
        pltpu.make_async_copy(
            rows0_v, acc.at[pl.ds(tid * ZROWS + i * CHUNK, CHUNK)], zsem).wait()
        return carry

    lax.fori_loop(0, ZROWS // CHUNK, _zdrain, 0)
    pltpu.make_async_copy(
        src_hbm.at[pl.ds(wid * E_PER_W, E_PER_W)], sidx_v, isem).wait()
    pltpu.make_async_copy(dst_hbm.at[wid], didx_v, isem).wait()
    plsc.subcore_barrier()

    def _gather(c, buf, sem):
        idx = sidx_v.at[pl.ds(c * CHUNK, CHUNK)]
        return pltpu.make_async_copy(sup_hbm.at[idx], buf, sem)

    def _scatter(c, buf):
        pltpu.sync_copy(buf, acc.at[didx_v.at[c]], add=True)

    _gather(0, rows0_v, gsem0).start()

    def _edge(i, carry):
        c0 = 2 * i
        _gather(c0 + 1, rows1_v, gsem1).start()
        _gather(c0, rows0_v, gsem0).wait()
        _scatter(c0, rows0_v)
        _gather(c0 + 2, rows0_v, gsem0).start()
        _gather(c0 + 1, rows1_v, gsem1).wait()
        _scatter(c0 + 1, rows1_v)
        return carry

    lax.fori_loop(0, (NCHUNK - 1) // 2, _edge, 0)
    _gather(NCHUNK - 1, rows0_v, gsem0).wait()
    _scatter(NCHUNK - 1, rows0_v)
    plsc.subcore_barrier()

    r0 = tid * ZROWS
    pltpu.sync_copy(acc.at[pl.ds(r0, ZROWS)], out_hbm.at[cid, pl.ds(r0, ZROWS)])


@functools.cache
def _sc_aggregate():
    return pl.kernel(
        _sc_body,
        out_type=jax.ShapeDtypeStruct((NC, ACC_ROWS, D), jnp.float32),
        mesh=plsc.VectorSubcoreMesh(
            core_axis_name="c", subcore_axis_name="s",
            num_cores=NC, num_subcores=NS,
        ),
        scratch_types=[
            pltpu.VMEM_SHARED((ACC_ROWS, D), jnp.float32),
            pltpu.VMEM((E_PER_W,), jnp.int32),
            pltpu.VMEM((NCHUNK, CHUNK), jnp.int32),
            pltpu.VMEM((CHUNK, D), jnp.float32),
            pltpu.VMEM((CHUNK, D), jnp.float32),
            pltpu.SemaphoreType.DMA,
            pltpu.SemaphoreType.DMA,
            pltpu.SemaphoreType.DMA,
            pltpu.SemaphoreType.DMA,
        ],
    )


def _combine_body(p_ref, o_ref):
    s = p_ref[0] + p_ref[1]
    o_ref[...] = jnp.where(s >= 0, s, 0.2 * s)


def _combine(partials):
    grid = 10
    rows = N // grid
    return pl.pallas_call(
        _combine_body,
        grid=(grid,),
        in_specs=[pl.BlockSpec((NC, rows, D), lambda i: (0, i, 0))],
        out_specs=pl.BlockSpec((rows, D), lambda i: (i, 0)),
        out_shape=jax.ShapeDtypeStruct((N, D), jnp.float32),
    )(partials)


def kernel(x, edge_index, W, b):
    support = _linear(x, W, b.reshape(1, D))
    dst3 = edge_index[1].reshape(NW, NCHUNK, CHUNK)
    partials = _sc_aggregate()(support, edge_index[0], dst3)
    return _combine(partials)


# speedup vs baseline: 13.3900x; 1.1546x over previous
import functools

import jax
import jax.numpy as jnp
from jax import lax
from jax.experimental import pallas as pl
from jax.experimental.pallas import tpu as pltpu
from jax.experimental.pallas import tpu_sc as plsc

N = 10000
E = 320000
D = 128

NC = 2
NS = 16
NW = NC * NS

E_PER_W = E // NW
CHUNK = 80
NCHUNK = E_PER_W // CHUNK

ACC_ROWS = 10240
ZROWS = 640


def _linear_body(x_ref, w_ref, b_ref, o_ref):
    o_ref[...] = lax.dot_general(
        x_ref[...], w_ref[...],
        dimension_numbers=(((1,), (1,)), ((), ())),
        preferred_element_type=jnp.float32,
    ) + b_ref[...]


def _linear(x, W, b2):
    grid = 10
    rows = N // grid
    return pl.pallas_call(
        _linear_body,
        grid=(grid,),
        in_specs=[
            pl.BlockSpec((rows, D), lambda i: (i, 0)),
            pl.BlockSpec((D, D), lambda i: (0, 0)),
            pl.BlockSpec((1, D), lambda i: (0, 0)),
        ],
        out_specs=pl.BlockSpec((rows, D), lambda i: (i, 0)),
        out_shape=jax.ShapeDtypeStruct((N, D), jnp.float32),
    )(x, W, b2)


NBUF = 4


def _sc_body(sup_hbm, src_hbm, dst_hbm, out_hbm, acc, rows, sidxb, didxb,
             gsem, ssem, isem, zsem):
    cid = lax.axis_index("c")
    sid = lax.axis_index("s")
    wid = cid * NS + sid
    tid = sid
    base = wid * E_PER_W

    def _start_idx(c, b):
        off = base + c * CHUNK
        pltpu.async_copy(src_hbm.at[pl.ds(off, CHUNK)], sidxb[b], isem[b])
        pltpu.async_copy(dst_hbm.at[pl.ds(off, CHUNK)], didxb[b], isem[b])

    def _wait_idx(c, b):
        off = base + c * CHUNK
        pltpu.make_async_copy(
            src_hbm.at[pl.ds(off, CHUNK)], sidxb[b], isem[b]).wait()
        pltpu.make_async_copy(
            dst_hbm.at[pl.ds(off, CHUNK)], didxb[b], isem[b]).wait()

    def _gather(b):
        return pltpu.make_async_copy(sup_hbm.at[sidxb[b]], rows[b], gsem[b])

    def _scatter_start(b):
        pltpu.async_copy(rows[b], acc.at[didxb[b]], ssem[b], add=True)

    def _scatter_wait(b):
        pltpu.make_async_copy(rows[b], acc.at[didxb[b]], ssem[b]).wait()

    def _zrow(r, carry):
        for j in range(8):
            rows[0][r, pl.ds(j * 16, 16)] = jnp.zeros((16,), jnp.float32)
        return carry

    lax.fori_loop(0, CHUNK, _zrow, 0)

    for c in range(3):
        _start_idx(c, c)

    def _zero(i, carry):
        pltpu.async_copy(
            rows[0], acc.at[pl.ds(tid * ZROWS + i * CHUNK, CHUNK)], zsem)
        return carry

    lax.fori_loop(0, ZROWS // CHUNK, _zero, 0)

    def _zdrain(i, carry):
        pltpu.make_async_copy(
            rows[0], acc.at[pl.ds(tid * ZROWS + i * CHUNK, CHUNK)], zsem).wait()
        return carry

    lax.fori_loop(0, ZROWS // CHUNK, _zdrain, 0)
    plsc.subcore_barrier()

    _wait_idx(0, 0)
    _gather(0).start()
    _wait_idx(1, 1)
    _gather(1).start()

    def _slot(c, b):
        bm1 = (b + 3) % NBUF
        bp2 = (b + 2) % NBUF

        @pl.when(c >= 1)
        def _():
            _scatter_wait(bm1)

        @pl.when(c <= NCHUNK - 3)
        def _():
            _wait_idx(c + 2, bp2)
            _gather(bp2).start()

        _gather(b).wait()
        _scatter_start(b)

        @pl.when(c <= NCHUNK - 4)
        def _():
            _start_idx(c + 3, bm1)

    def _outer(i, carry):
        c0 = 4 * i
        for b in range(4):
            _slot(c0 + b, b)
        return carry

    lax.fori_loop(0, (NCHUNK - 1) // 4, _outer, 0)
    _scatter_wait((NCHUNK - 2) % NBUF)
    _gather((NCHUNK - 1) % NBUF).wait()
    _scatter_start((NCHUNK - 1) % NBUF)
    _scatter_wait((NCHUNK - 1) % NBUF)
    plsc.subcore_barrier()

    r0 = tid * ZROWS
    pltpu.sync_copy(acc.at[pl.ds(r0, ZROWS)], out_hbm.at[cid, pl.ds(r0, ZROWS)])


@functools.cache
def _sc_aggregate():
    return pl.kernel(
        _sc_body,
        out_type=jax.ShapeDtypeStruct((NC, ACC_ROWS, D), jnp.float32),
        mesh=plsc.VectorSubcoreMesh(
            core_axis_name="c", subcore_axis_name="s",
            num_cores=NC, num_subcores=NS,
        ),
        scratch_types=[
            pltpu.VMEM_SHARED((ACC_ROWS, D), jnp.float32),
            [pltpu.VMEM((CHUNK, D), jnp.float32) for _ in range(NBUF)],
            [pltpu.VMEM((CHUNK,), jnp.int32) for _ in range(NBUF)],
            [pltpu.VMEM((CHUNK,), jnp.int32) for _ in range(NBUF)],
            [pltpu.SemaphoreType.DMA for _ in range(NBUF)],
            [pltpu.SemaphoreType.DMA for _ in range(NBUF)],
            [pltpu.SemaphoreType.DMA for _ in range(NBUF)],
            pltpu.SemaphoreType.DMA,
        ],
    )


def _combine_body(p_ref, o_ref):
    s = p_ref[0] + p_ref[1]
    o_ref[...] = jnp.where(s >= 0, s, 0.2 * s)


def _combine(partials):
    grid = 10
    rows = N // grid
    return pl.pallas_call(
        _combine_body,
        grid=(grid,),
        in_specs=[pl.BlockSpec((NC, rows, D), lambda i: (0, i, 0))],
        out_specs=pl.BlockSpec((rows, D), lambda i: (i, 0)),
        out_shape=jax.ShapeDtypeStruct((N, D), jnp.float32),
    )(partials)


def kernel(x, edge_index, W, b):
    support = _linear(x, W, b.reshape(1, D))
    partials = _sc_aggregate()(support, edge_index[0], edge_index[1])
    return _combine(partials)


# speedup vs baseline: 13.9452x; 1.0415x over previous
import functools

import jax
import jax.numpy as jnp
from jax import lax
from jax.experimental import pallas as pl
from jax.experimental.pallas import tpu as pltpu
from jax.experimental.pallas import tpu_sc as plsc

N = 10000
E = 320000
D = 128

NC = 2
NS = 16
NW = NC * NS

CHUNK = 128
TCHUNKS = E // CHUNK
BASE_CNT = TCHUNKS // NW
EXTRA = TCHUNKS - BASE_CNT * NW

NBUF = 3

ACC_ROWS = 10112
ZROWS = 632


def _linear_body(x_ref, w_ref, b_ref, o_ref):
    o_ref[...] = lax.dot_general(
        x_ref[...], w_ref[...],
        dimension_numbers=(((1,), (1,)), ((), ())),
        preferred_element_type=jnp.float32,
    ) + b_ref[...]


def _linear(x, W, b2):
    grid = 10
    rows = N // grid
    return pl.pallas_call(
        _linear_body,
        grid=(grid,),
        in_specs=[
            pl.BlockSpec((rows, D), lambda i: (i, 0)),
            pl.BlockSpec((D, D), lambda i: (0, 0)),
            pl.BlockSpec((1, D), lambda i: (0, 0)),
        ],
        out_specs=pl.BlockSpec((rows, D), lambda i: (i, 0)),
        out_shape=jax.ShapeDtypeStruct((N, D), jnp.float32),
    )(x, W, b2)


def _sc_body(sup_hbm, ei_hbm, out_hbm, acc, rows, idxb, gsem, ssem, isem,
             zsem):
    cid = lax.axis_index("c")
    sid = lax.axis_index("s")
    wid = cid * NS + sid
    tid = sid
    start = wid * BASE_CNT + jnp.minimum(wid, EXTRA)

    def _start_idx(c, b):
        pltpu.async_copy(
            ei_hbm.at[:, pl.ds((start + c) * CHUNK, CHUNK)], idxb[b], isem[b])

    def _wait_idx(c, b):
        pltpu.make_async_copy(
            ei_hbm.at[:, pl.ds((start + c) * CHUNK, CHUNK)], idxb[b],
            isem[b]).wait()

    def _gather(b):
        return pltpu.make_async_copy(sup_hbm.at[idxb[b].at[0]], rows[b],
                                     gsem[b])

    def _scatter_start(b):
        pltpu.async_copy(rows[b], acc.at[idxb[b].at[1]], ssem[b], add=True)

    def _scatter_wait(b):
        pltpu.make_async_copy(rows[b], acc.at[idxb[b].at[1]], ssem[b]).wait()

    def _zrow(r, carry):
        for j in range(8):
            rows[0][r, pl.ds(j * 16, 16)] = jnp.zeros((16,), jnp.float32)
        return carry

    lax.fori_loop(0, CHUNK, _zrow, 0)

    _start_idx(0, 0)
    _start_idx(1, 1)

    z0 = tid * ZROWS
    for j in range(4):
        pltpu.async_copy(rows[0], acc.at[pl.ds(z0 + j * 128, 128)], zsem)
    pltpu.async_copy(
        rows[0].at[pl.ds(0, ZROWS - 512)],
        acc.at[pl.ds(z0 + 512, ZROWS - 512)], zsem)
    for j in range(4):
        pltpu.make_async_copy(
            rows[0], acc.at[pl.ds(z0 + j * 128, 128)], zsem).wait()
    pltpu.make_async_copy(
        rows[0].at[pl.ds(0, ZROWS - 512)],
        acc.at[pl.ds(z0 + 512, ZROWS - 512)], zsem).wait()
    plsc.subcore_barrier()

    _wait_idx(0, 0)
    _gather(0).start()

    def _slot(c, b):
        bm1 = (b + 2) % NBUF
        bp1 = (b + 1) % NBUF

        @pl.when(c >= 1)
        def _():
            _scatter_wait(bm1)

        @pl.when(c <= BASE_CNT - 2)
        def _():
            _wait_idx(c + 1, bp1)
            _gather(bp1).start()

        _gather(b).wait()
        _scatter_start(b)

        @pl.when(c <= BASE_CNT - 3)
        def _():
            _start_idx(c + 2, bm1)

    def _outer(i, carry):
        c0 = NBUF * i
        for b in range(NBUF):
            _slot(c0 + b, b)
        return carry

    lax.fori_loop(0, BASE_CNT // NBUF, _outer, 0)
    _scatter_wait((BASE_CNT - 1) % NBUF)

    @pl.when(wid < EXTRA)
    def _():
        off = (start + BASE_CNT) * CHUNK
        pltpu.sync_copy(ei_hbm.at[:, pl.ds(off, CHUNK)], idxb[0])
        g = _gather(0)
        g.start()
        g.wait()
        pltpu.sync_copy(rows[0], acc.at[idxb[0].at[1]], add=True)

    plsc.subcore_barrier()

    pltpu.sync_copy(acc.at[pl.ds(z0, ZROWS)], out_hbm.at[cid, pl.ds(z0, ZROWS)])


@functools.cache
def _sc_aggregate():
    return pl.kernel(
        _sc_body,
        out_type=jax.ShapeDtypeStruct((NC, ACC_ROWS, D), jnp.float32),
        mesh=plsc.VectorSubcoreMesh(
            core_axis_name="c", subcore_axis_name="s",
            num_cores=NC, num_subcores=NS,
        ),
        scratch_types=[
            pltpu.VMEM_SHARED((ACC_ROWS, D), jnp.float32),
            [pltpu.VMEM((CHUNK, D), jnp.float32) for _ in range(NBUF)],
            [pltpu.VMEM((2, CHUNK), jnp.int32) for _ in range(NBUF)],
            [pltpu.SemaphoreType.DMA for _ in range(NBUF)],
            [pltpu.SemaphoreType.DMA for _ in range(NBUF)],
            [pltpu.SemaphoreType.DMA for _ in range(NBUF)],
            pltpu.SemaphoreType.DMA,
        ],
    )


def _combine_body(p_ref, o_ref):
    s = p_ref[0] + p_ref[1]
    o_ref[...] = jnp.where(s >= 0, s, 0.2 * s)


def _combine(partials):
    grid = 10
    rows = N // grid
    return pl.pallas_call(
        _combine_body,
        grid=(grid,),
        in_specs=[pl.BlockSpec((NC, rows, D), lambda i: (0, i, 0))],
        out_specs=pl.BlockSpec((rows, D), lambda i: (i, 0)),
        out_shape=jax.ShapeDtypeStruct((N, D), jnp.float32),
    )(partials)


def kernel(x, edge_index, W, b):
    support = _linear(x, W, b.reshape(1, D))
    partials = _sc_aggregate()(support, edge_index)
    return _combine(partials)
